# baseline (speedup 1.0000x reference)
import jax
import jax.numpy as jnp
from jax.experimental import pallas as pl
from jax.experimental.pallas import tpu as pltpu

_BLOCK_T = 1024
_DEPTH = 4


def _router_body(x_hbm, w_ref, b_ref, o_ref, xbuf, sems):
    n_blocks = x_hbm.shape[0] // _BLOCK_T

    def copy_in(step, slot):
        return pltpu.make_async_copy(
            x_hbm.at[pl.ds(step * _BLOCK_T, _BLOCK_T), :],
            xbuf.at[slot],
            sems.at[slot],
        )

    for s in range(_DEPTH):
        copy_in(s, s).start()

    w = w_ref[...]
    b = b_ref[...][None, :]

    for i in range(n_blocks):
        slot = i % _DEPTH
        copy_in(i, slot).wait()
        o_ref[i * _BLOCK_T : (i + 1) * _BLOCK_T, :] = (
            jnp.dot(xbuf[slot], w, preferred_element_type=jnp.float32) + b
        )
        if i + _DEPTH < n_blocks:
            copy_in(i + _DEPTH, slot).start()


def kernel(x, W, b):
    n_tokens, d_model = x.shape
    n_experts = W.shape[1]
    return pl.pallas_call(
        _router_body,
        grid=(1,),
        in_specs=[
            pl.BlockSpec(memory_space=pltpu.MemorySpace.HBM),
            pl.BlockSpec((d_model, n_experts), lambda i: (0, 0)),
            pl.BlockSpec((n_experts,), lambda i: (0,)),
        ],
        out_specs=pl.BlockSpec((n_tokens, n_experts), lambda i: (0, 0)),
        out_shape=jax.ShapeDtypeStruct((n_tokens, n_experts), jnp.float32),
        scratch_shapes=[
            pltpu.VMEM((_DEPTH, _BLOCK_T, d_model), jnp.float32),
            pltpu.SemaphoreType.DMA((_DEPTH,)),
        ],
        compiler_params=pltpu.CompilerParams(
            dimension_semantics=("arbitrary",),
        ),
    )(x, W, b)


# speedup vs baseline: 1.3390x; 1.3390x over previous
import jax
import jax.numpy as jnp
from jax import lax
from jax.experimental import pallas as pl
from jax.experimental.pallas import tpu as pltpu

_BLOCK_T = 1024


def _router_body(wt_ref, b_ref, x_ref, o_ref):
    o_ref[...] = (
        lax.dot_general(
            wt_ref[...],
            x_ref[...],
            ((( 1,), (1,)), ((), ())),
            preferred_element_type=jnp.float32,
        )
        + b_ref[...][:, None]
    )


def kernel(x, W, b):
    n_tokens, d_model = x.shape
    n_experts = W.shape[1]
    out_t = pl.pallas_call(
        _router_body,
        grid=(n_tokens // _BLOCK_T,),
        in_specs=[
            pl.BlockSpec((n_experts, d_model), lambda i: (0, 0)),
            pl.BlockSpec((n_experts,), lambda i: (0,)),
            pl.BlockSpec((_BLOCK_T, d_model), lambda i: (i, 0)),
        ],
        out_specs=pl.BlockSpec((n_experts, _BLOCK_T), lambda i: (0, i)),
        out_shape=jax.ShapeDtypeStruct((n_experts, n_tokens), jnp.float32),
        compiler_params=pltpu.CompilerParams(
            dimension_semantics=("arbitrary",),
        ),
    )(W.T, b, x)
    return out_t.T
